# baseline (speedup 1.0000x reference)
import functools

import jax
import jax.numpy as jnp
from jax import lax
from jax.experimental import pallas as pl
from jax.experimental.pallas import tpu as pltpu


def _fused_attn_kernel(x_ref, wqkv_ref, wproj_ref, bproj_ref, o_ref,
                       merged_scr, wqkv_bf_scr, wproj_bf_scr,
                       *, num_heads, rows, scale):
    n, c = x_ref.shape[1], x_ref.shape[2]
    hd = c // num_heads
    group = 4
    ones_col = jnp.ones((n, 1), dtype=jnp.bfloat16)

    @pl.when(pl.program_id(0) == 0)
    def _prep_weights():
        wqkv_bf_scr[:, :c] = (wqkv_ref[:, :c] * scale).astype(jnp.bfloat16)
        wqkv_bf_scr[:, c:] = wqkv_ref[:, c:].astype(jnp.bfloat16)
        wproj_bf_scr[...] = wproj_ref[...].astype(jnp.bfloat16)

    for r in range(rows):
        x = x_ref[r].astype(jnp.bfloat16)
        qkv_bf = jnp.dot(x, wqkv_bf_scr[...],
                         preferred_element_type=jnp.float32
                         ).astype(jnp.bfloat16)

        out = bproj_ref[...].astype(jnp.float32)

        def head_ctx(h, qkv_bf=qkv_bf):
            qh = qkv_bf[:, h * hd:(h + 1) * hd]
            kh = qkv_bf[:, c + h * hd:c + (h + 1) * hd]
            vh = qkv_bf[:, 2 * c + h * hd:2 * c + (h + 1) * hd]
            s = lax.dot_general(qh, kh, (((1,), (1,)), ((), ())),
                                preferred_element_type=jnp.float32)
            m = jnp.max(s, axis=-1, keepdims=True)
            p = jnp.exp((s - m).astype(jnp.bfloat16))
            v_aug = jnp.concatenate([vh, ones_col], axis=-1)
            pv = jnp.dot(p, v_aug,
                         preferred_element_type=jnp.float32)
            inv_l = pl.reciprocal(pv[:, hd:hd + 1], approx=True)
            return (pv[:, :hd] * inv_l).astype(jnp.bfloat16)

        r0 = r * n
        for g in range(num_heads // group):
            for hp in range(group // 2):
                h = g * group + 2 * hp
                merged_scr[r0:r0 + n, h * hd:(h + 2) * hd] = jnp.concatenate(
                    [head_ctx(h), head_ctx(h + 1)], axis=-1)
            gs, ge = g * group * hd, (g + 1) * group * hd
            out = out + jnp.dot(merged_scr[r0:r0 + n, gs:ge],
                                wproj_bf_scr[gs:ge, :],
                                preferred_element_type=jnp.float32)

        o_ref[r] = out.astype(o_ref.dtype)


def kernel(x, wqkv, wproj, bproj):
    B, N, C = x.shape
    H = 12
    hd = C // H
    scale = hd ** (-0.5)
    bproj2d = bproj.reshape(1, C).astype(jnp.float32)

    rows = 4
    return pl.pallas_call(
        functools.partial(_fused_attn_kernel, num_heads=H, rows=rows,
                          scale=scale),
        out_shape=jax.ShapeDtypeStruct((B, N, C), x.dtype),
        grid=(B // rows,),
        in_specs=[
            pl.BlockSpec((rows, N, C), lambda b: (b, 0, 0)),
            pl.BlockSpec((C, 3 * C), lambda b: (0, 0)),
            pl.BlockSpec((C, C), lambda b: (0, 0)),
            pl.BlockSpec((1, C), lambda b: (0, 0)),
        ],
        out_specs=pl.BlockSpec((rows, N, C), lambda b: (b, 0, 0)),
        scratch_shapes=[
            pltpu.VMEM((rows * N, C), jnp.bfloat16),
            pltpu.VMEM((C, 3 * C), jnp.bfloat16),
            pltpu.VMEM((C, C), jnp.bfloat16),
        ],
        compiler_params=pltpu.CompilerParams(
            dimension_semantics=("arbitrary",),
            vmem_limit_bytes=56 * 1024 * 1024),
    )(x, wqkv, wproj, bproj2d)


# speedup vs baseline: 1.0087x; 1.0087x over previous
import functools

import jax
import jax.numpy as jnp
from jax import lax
from jax.experimental import pallas as pl
from jax.experimental.pallas import tpu as pltpu


def _fused_attn_kernel(x_ref, wqkv_ref, wproj_ref, bproj_ref, o_ref,
                       merged_scr, wqkv_bf_scr, wproj_bf_scr,
                       *, num_heads, rows, scale):
    n, c = x_ref.shape[1], x_ref.shape[2]
    hd = c // num_heads
    group = 4
    ones_col = jnp.ones((n, 1), dtype=jnp.bfloat16)

    @pl.when(pl.program_id(0) == 0)
    def _prep_weights():
        wqkv_bf_scr[:, :c] = (wqkv_ref[:, :c] * scale).astype(jnp.bfloat16)
        wqkv_bf_scr[:, c:] = wqkv_ref[:, c:].astype(jnp.bfloat16)
        wproj_bf_scr[...] = wproj_ref[...].astype(jnp.bfloat16)

    for r in range(rows):
        x = x_ref[r].astype(jnp.bfloat16)
        qkv_bf = jnp.dot(x, wqkv_bf_scr[...],
                         preferred_element_type=jnp.float32
                         ).astype(jnp.bfloat16)

        out = bproj_ref[...].astype(jnp.float32)

        def head_ctx(h, qkv_bf=qkv_bf):
            qh = qkv_bf[:, h * hd:(h + 1) * hd]
            kh = qkv_bf[:, c + h * hd:c + (h + 1) * hd]
            vh = qkv_bf[:, 2 * c + h * hd:2 * c + (h + 1) * hd]
            s = lax.dot_general(qh, kh, (((1,), (1,)), ((), ())),
                                preferred_element_type=jnp.float32)
            m = jnp.max(s, axis=-1, keepdims=True)
            p = jnp.exp((s - m).astype(jnp.bfloat16))
            v_aug = jnp.concatenate([vh, ones_col], axis=-1)
            pv = jnp.dot(p, v_aug,
                         preferred_element_type=jnp.float32)
            inv_l = pl.reciprocal(pv[:, hd:hd + 1], approx=True)
            return (pv[:, :hd] * inv_l).astype(jnp.bfloat16)

        r0 = r * n
        for g in range(num_heads // group):
            for hp in range(group // 2):
                h = g * group + 2 * hp
                merged_scr[r0:r0 + n, h * hd:(h + 2) * hd] = jnp.concatenate(
                    [head_ctx(h), head_ctx(h + 1)], axis=-1)
            gs, ge = g * group * hd, (g + 1) * group * hd
            out = out + jnp.dot(merged_scr[r0:r0 + n, gs:ge],
                                wproj_bf_scr[gs:ge, :],
                                preferred_element_type=jnp.float32)

        o_ref[r] = out.astype(o_ref.dtype)


def kernel(x, wqkv, wproj, bproj):
    B, N, C = x.shape
    H = 12
    hd = C // H
    scale = hd ** (-0.5)
    bproj2d = bproj.reshape(1, C).astype(jnp.float32)

    rows = 2
    return pl.pallas_call(
        functools.partial(_fused_attn_kernel, num_heads=H, rows=rows,
                          scale=scale),
        out_shape=jax.ShapeDtypeStruct((B, N, C), x.dtype),
        grid=(B // rows,),
        in_specs=[
            pl.BlockSpec((rows, N, C), lambda b: (b, 0, 0)),
            pl.BlockSpec((C, 3 * C), lambda b: (0, 0)),
            pl.BlockSpec((C, C), lambda b: (0, 0)),
            pl.BlockSpec((1, C), lambda b: (0, 0)),
        ],
        out_specs=pl.BlockSpec((rows, N, C), lambda b: (b, 0, 0)),
        scratch_shapes=[
            pltpu.VMEM((rows * N, C), jnp.bfloat16),
            pltpu.VMEM((C, 3 * C), jnp.bfloat16),
            pltpu.VMEM((C, C), jnp.bfloat16),
        ],
        compiler_params=pltpu.CompilerParams(
            dimension_semantics=("arbitrary",),
            vmem_limit_bytes=56 * 1024 * 1024),
    )(x, wqkv, wproj, bproj2d)
